# baseline (speedup 1.0000x reference)
import functools

import jax
import jax.numpy as jnp
from jax import lax
from jax.experimental import pallas as pl
from jax.experimental.pallas import tpu as pltpu
from jax.experimental.pallas import tpu_sc as plsc

VOCAB = 1000000
D = 64
B = 4096
S = 200
NC = 2
NS = 16
NW = NC * NS
BBLK = B // NW
NG = BBLK // 16


def _sc_body(idx_hbm, tbl_hbm, pos_hbm, out_hbm,
             idx_v, i0, i1, g0, g1, o0, o1, pos_v, sem0, sem1):
    c = lax.axis_index("c")
    s = lax.axis_index("s")
    w = s * NC + c
    pltpu.sync_copy(idx_hbm.at[:, w], idx_v)
    pltpu.sync_copy(pos_hbm, pos_v)

    iota = lax.iota(jnp.int32, 16)

    def start(ibuf, gbuf, sem, p):
        def shift_body(q):
            ibuf[pl.ds(q * 16, 16)] = (
                lax.shift_right_logical(idx_v[p, pl.ds(q * 16, 16)], 1)
            )
        plsc.parallel_loop(0, NG, unroll=8)(shift_body)
        pltpu.async_copy(tbl_hbm.at[ibuf], gbuf, sem)

    def finish(ibuf, gbuf, obuf, sem, p):
        pltpu.make_async_copy(tbl_hbm.at[pl.ds(0, BBLK)], gbuf, sem).wait()

        def grp_body(q, carry):
            lane = iota + q * 16
            parity = idx_v[p, pl.ds(q * 16, 16)] & 1
            cbase = parity * 64
            for dh in range(4):
                pvec = pos_v[p, pl.ds(dh * 16, 16)]
                for dl in range(16):
                    d = dh * 16 + dl
                    vals = plsc.load_gather(gbuf, [lane, cbase + d])
                    obuf[d // 8, d % 8, pl.ds(q * 16, 16)] = vals + pvec[dl]
            return carry

        lax.fori_loop(0, NG, grp_body, 0)
        pltpu.sync_copy(obuf, out_hbm.at[p, :, w])

    start(i0, g0, sem0, 0)

    def pair_body(h, carry):
        p0 = 2 * h
        start(i1, g1, sem1, p0 + 1)
        finish(i0, g0, o0, sem0, p0)
        start(i0, g0, sem0, p0 + 2)
        finish(i1, g1, o1, sem1, p0 + 1)
        return carry

    lax.fori_loop(0, S // 2 - 1, pair_body, 0)
    start(i1, g1, sem1, S - 1)
    finish(i0, g0, o0, sem0, S - 2)
    finish(i1, g1, o1, sem1, S - 1)


@jax.jit
def _run(idx5, tbl2, pos2d):
    mesh = plsc.VectorSubcoreMesh(core_axis_name="c", subcore_axis_name="s")
    f = functools.partial(
        pl.kernel,
        out_type=jax.ShapeDtypeStruct((S, 8, NW, 8, 128), jnp.float32),
        mesh=mesh,
        scratch_types=[
            pltpu.VMEM((S, BBLK), jnp.int32),
            pltpu.VMEM((BBLK,), jnp.int32),
            pltpu.VMEM((BBLK,), jnp.int32),
            pltpu.VMEM((BBLK, 128), jnp.float32),
            pltpu.VMEM((BBLK, 128), jnp.float32),
            pltpu.VMEM((8, 8, BBLK), jnp.float32),
            pltpu.VMEM((8, 8, BBLK), jnp.float32),
            pltpu.VMEM((S, D), jnp.float32),
            pltpu.SemaphoreType.DMA,
            pltpu.SemaphoreType.DMA,
        ],
        compiler_params=pltpu.CompilerParams(use_tc_tiling_on_sc=False, needs_layout_passes=False),
    )(_sc_body)
    return f(idx5, tbl2, pos2d)


def kernel(INPUT, embedding_table, positional_encoding):
    idx5 = INPUT.T.reshape(S, NW, 128)
    tbl2 = embedding_table.reshape(VOCAB // 2, 2 * D)
    pos2d = positional_encoding[0, :S, :]
    out5 = _run(idx5, tbl2, pos2d)
    return out5.transpose(2, 4, 0, 1, 3).reshape(B, S, D)


# speedup vs baseline: 2.4194x; 2.4194x over previous
import functools

import jax
import jax.numpy as jnp
from jax import lax
from jax.experimental import pallas as pl
from jax.experimental.pallas import tpu as pltpu
from jax.experimental.pallas import tpu_sc as plsc

VOCAB = 1000000
D = 64
B = 4096
S = 200
NC = 2
NS = 16
NW = NC * NS
BBLK = B // NW
OP = 129


def _sc_body(idx_hbm, tbl_hbm, pos_hbm, out_hbm,
             idx_v, g0, g1, o0, o1, pos_v, sem0, sem1):
    c = lax.axis_index("c")
    s = lax.axis_index("s")
    w = s * NC + c
    pltpu.sync_copy(idx_hbm.at[:, w], idx_v)
    pltpu.sync_copy(pos_hbm, pos_v)

    iota = lax.iota(jnp.int32, 16)
    dl_vec = iota & 7
    dt_vecs = [2 * dh + (iota >> 3) for dh in range(4)]

    def start(gbuf, sem, p):
        pltpu.async_copy(tbl_hbm.at[idx_v.at[p]], gbuf, sem)

    def finish(gbuf, obuf, sem, p):
        pltpu.make_async_copy(tbl_hbm.at[pl.ds(0, BBLK)], gbuf, sem).wait()
        pv = [pos_v[p, pl.ds(16 * k, 16)] for k in range(4)]

        def b_body(b):
            bfull = iota * 0 + b
            for dh in range(4):
                vals = gbuf[b, pl.ds(16 * dh, 16)] + pv[dh]
                plsc.store_scatter(obuf, [dt_vecs[dh], dl_vec, bfull], vals)

        plsc.parallel_loop(0, BBLK, unroll=4)(b_body)
        for dt in range(8):
            pltpu.sync_copy(obuf.at[dt, :, pl.ds(0, BBLK)],
                            out_hbm.at[p, dt, w])

    start(g0, sem0, 0)

    def pair_body(h, carry):
        p0 = 2 * h
        start(g1, sem1, p0 + 1)
        finish(g0, o0, sem0, p0)
        start(g0, sem0, p0 + 2)
        finish(g1, o1, sem1, p0 + 1)
        return carry

    lax.fori_loop(0, S // 2 - 1, pair_body, 0)
    start(g1, sem1, S - 1)
    finish(g0, o0, sem0, S - 2)
    finish(g1, o1, sem1, S - 1)


@jax.jit
def _run(idx5, tbl, pos2d):
    mesh = plsc.VectorSubcoreMesh(core_axis_name="c", subcore_axis_name="s")
    f = functools.partial(
        pl.kernel,
        out_type=jax.ShapeDtypeStruct((S, 8, NW, 8, 128), jnp.float32),
        mesh=mesh,
        scratch_types=[
            pltpu.VMEM((S, BBLK), jnp.int32),
            pltpu.VMEM((BBLK, D), jnp.float32),
            pltpu.VMEM((BBLK, D), jnp.float32),
            pltpu.VMEM((8, 8, OP), jnp.float32),
            pltpu.VMEM((8, 8, OP), jnp.float32),
            pltpu.VMEM((S, D), jnp.float32),
            pltpu.SemaphoreType.DMA,
            pltpu.SemaphoreType.DMA,
        ],
        compiler_params=pltpu.CompilerParams(
            use_tc_tiling_on_sc=False, needs_layout_passes=False),
    )(_sc_body)
    return f(idx5, tbl, pos2d)


def kernel(INPUT, embedding_table, positional_encoding):
    idx5 = INPUT.T.reshape(S, NW, 128)
    pos2d = positional_encoding[0, :S, :]
    out5 = _run(idx5, embedding_table, pos2d)
    return out5.transpose(2, 4, 0, 1, 3).reshape(B, S, D)


# speedup vs baseline: 2.7275x; 1.1274x over previous
import functools

import jax
import jax.numpy as jnp
from jax import lax
from jax.experimental import pallas as pl
from jax.experimental.pallas import tpu as pltpu
from jax.experimental.pallas import tpu_sc as plsc

VOCAB = 1000000
D = 64
B = 4096
S = 200
NC = 2
NS = 16
NW = NC * NS
BBLK = B // NW
OP = 129
HV = VOCAB // 2


def _sc_body(idx_hbm, tbl_hbm, pos_hbm, out_hbm,
             idx_v, g0, g1, o0, o1, pos_v, sem0, sem1):
    c = lax.axis_index("c")
    s = lax.axis_index("s")
    w = s * NC + c
    pltpu.sync_copy(idx_hbm.at[:, w], idx_v)
    pltpu.sync_copy(pos_hbm, pos_v)

    def xform_body(p):
        for k in range(8):
            v = idx_v[p, pl.ds(16 * k, 16)]
            idx_v[p, pl.ds(16 * k, 16)] = (
                v + v - jnp.where(v >= HV, 2 * HV - 1, 0)
            )

    plsc.parallel_loop(0, S, unroll=2)(xform_body)

    iota = lax.iota(jnp.int32, 16)
    dl_vec = iota & 7
    dt_vecs = [2 * dh + (iota >> 3) for dh in range(4)]

    def start(gbuf, sem, p):
        pltpu.async_copy(tbl_hbm.at[idx_v.at[p]], gbuf, sem)

    def finish(gbuf, obuf, sem, p):
        pltpu.make_async_copy(tbl_hbm.at[pl.ds(0, BBLK)], gbuf, sem).wait()
        pv = [pos_v[p, pl.ds(16 * k, 16)] for k in range(4)]

        def b_body(b):
            bfull = iota * 0 + b
            for dh in range(4):
                vals = gbuf[b, pl.ds(16 * dh, 16)] + pv[dh]
                plsc.store_scatter(obuf, [dt_vecs[dh], dl_vec, bfull], vals)

        plsc.parallel_loop(0, BBLK, unroll=4)(b_body)
        for dt in range(8):
            pltpu.sync_copy(obuf.at[dt, :, pl.ds(0, BBLK)],
                            out_hbm.at[p, dt, w])

    start(g0, sem0, 0)

    def pair_body(h, carry):
        p0 = 2 * h
        start(g1, sem1, p0 + 1)
        finish(g0, o0, sem0, p0)
        start(g0, sem0, p0 + 2)
        finish(g1, o1, sem1, p0 + 1)
        return carry

    lax.fori_loop(0, S // 2 - 1, pair_body, 0)
    start(g1, sem1, S - 1)
    finish(g0, o0, sem0, S - 2)
    finish(g1, o1, sem1, S - 1)


@jax.jit
def _run(idx5, tbl, pos2d):
    mesh = plsc.VectorSubcoreMesh(core_axis_name="c", subcore_axis_name="s")
    f = functools.partial(
        pl.kernel,
        out_type=jax.ShapeDtypeStruct((S, 8, NW, 8, 128), jnp.float32),
        mesh=mesh,
        scratch_types=[
            pltpu.VMEM((S, BBLK), jnp.int32),
            pltpu.VMEM((BBLK, D), jnp.float32),
            pltpu.VMEM((BBLK, D), jnp.float32),
            pltpu.VMEM((8, 8, OP), jnp.float32),
            pltpu.VMEM((8, 8, OP), jnp.float32),
            pltpu.VMEM((S, D), jnp.float32),
            pltpu.SemaphoreType.DMA,
            pltpu.SemaphoreType.DMA,
        ],
        compiler_params=pltpu.CompilerParams(
            use_tc_tiling_on_sc=False, needs_layout_passes=False),
    )(_sc_body)
    return f(idx5, tbl, pos2d)


def kernel(INPUT, embedding_table, positional_encoding):
    idx5 = INPUT.T.reshape(S, NW, 128)
    pos2d = positional_encoding[0, :S, :]
    tbl_rm = jnp.concatenate(
        [embedding_table[:HV], embedding_table[HV:]], axis=1
    ).reshape(VOCAB, D)
    out5 = _run(idx5, tbl_rm, pos2d)
    return out5.transpose(2, 4, 0, 1, 3).reshape(B, S, D)


# speedup vs baseline: 2.9102x; 1.0670x over previous
import functools

import jax
import jax.numpy as jnp
from jax import lax
from jax.experimental import pallas as pl
from jax.experimental.pallas import tpu as pltpu
from jax.experimental.pallas import tpu_sc as plsc

VOCAB = 1000000
D = 64
B = 4096
S = 200
NC = 2
NS = 16
NW = NC * NS
BBLK = B // NW
OP = 129
HV = VOCAB // 2


def _sc_body(idx_hbm, tbl_hbm, pos_hbm, out_hbm,
             idx_v, g0, g1, o0, o1, pos_v, sem0, sem1, osem0, osem1):
    c = lax.axis_index("c")
    s = lax.axis_index("s")
    w = s * NC + c
    pltpu.sync_copy(idx_hbm.at[:, w], idx_v)
    pltpu.sync_copy(pos_hbm, pos_v)

    def xform_body(p):
        for k in range(8):
            v = idx_v[p, pl.ds(16 * k, 16)]
            idx_v[p, pl.ds(16 * k, 16)] = (
                v + v - jnp.where(v >= HV, 2 * HV - 1, 0)
            )

    plsc.parallel_loop(0, S, unroll=2)(xform_body)

    iota = lax.iota(jnp.int32, 16)
    dl_vec = iota & 7
    dt_vecs = [2 * dh + (iota >> 3) for dh in range(4)]

    def start(gbuf, sem, p):
        pltpu.async_copy(tbl_hbm.at[idx_v.at[p]], gbuf, sem)

    def drain_out(obuf, osem, p):
        for dt in range(8):
            pltpu.make_async_copy(obuf.at[dt, :, pl.ds(0, BBLK)],
                                  out_hbm.at[p, dt, w], osem).wait()

    def finish(gbuf, obuf, sem, osem, p):
        pltpu.make_async_copy(tbl_hbm.at[pl.ds(0, BBLK)], gbuf, sem).wait()

        @pl.when(p >= 2)
        def _():
            drain_out(obuf, osem, p)

        pv = [pos_v[p, pl.ds(16 * k, 16)] for k in range(4)]

        def b_body(b):
            bfull = iota * 0 + b
            for dh in range(4):
                vals = gbuf[b, pl.ds(16 * dh, 16)] + pv[dh]
                plsc.store_scatter(obuf, [dt_vecs[dh], dl_vec, bfull], vals)

        plsc.parallel_loop(0, BBLK, unroll=4)(b_body)
        for dt in range(8):
            pltpu.async_copy(obuf.at[dt, :, pl.ds(0, BBLK)],
                             out_hbm.at[p, dt, w], osem)

    start(g0, sem0, 0)

    def pair_body(h, carry):
        p0 = 2 * h
        start(g1, sem1, p0 + 1)
        finish(g0, o0, sem0, osem0, p0)
        start(g0, sem0, p0 + 2)
        finish(g1, o1, sem1, osem1, p0 + 1)
        return carry

    lax.fori_loop(0, S // 2 - 1, pair_body, 0)
    start(g1, sem1, S - 1)
    finish(g0, o0, sem0, osem0, S - 2)
    finish(g1, o1, sem1, osem1, S - 1)
    drain_out(o0, osem0, S - 2)
    drain_out(o1, osem1, S - 1)


@jax.jit
def _run(idx5, tbl, pos2d):
    mesh = plsc.VectorSubcoreMesh(core_axis_name="c", subcore_axis_name="s")
    f = functools.partial(
        pl.kernel,
        out_type=jax.ShapeDtypeStruct((S, 8, NW, 8, 128), jnp.float32),
        mesh=mesh,
        scratch_types=[
            pltpu.VMEM((S, BBLK), jnp.int32),
            pltpu.VMEM((BBLK, D), jnp.float32),
            pltpu.VMEM((BBLK, D), jnp.float32),
            pltpu.VMEM((8, 8, OP), jnp.float32),
            pltpu.VMEM((8, 8, OP), jnp.float32),
            pltpu.VMEM((S, D), jnp.float32),
            pltpu.SemaphoreType.DMA,
            pltpu.SemaphoreType.DMA,
            pltpu.SemaphoreType.DMA,
            pltpu.SemaphoreType.DMA,
        ],
        compiler_params=pltpu.CompilerParams(
            use_tc_tiling_on_sc=False, needs_layout_passes=False),
    )(_sc_body)
    return f(idx5, tbl, pos2d)


def kernel(INPUT, embedding_table, positional_encoding):
    idx5 = INPUT.T.reshape(S, NW, 128)
    pos2d = positional_encoding[0, :S, :]
    tbl_rm = jnp.concatenate(
        [embedding_table[:HV], embedding_table[HV:]], axis=1
    ).reshape(VOCAB, D)
    out5 = _run(idx5, tbl_rm, pos2d)
    return out5.transpose(2, 4, 0, 1, 3).reshape(B, S, D)
